# baseline (speedup 1.0000x reference)
import functools

import jax
import jax.numpy as jnp
import numpy as np
from jax import lax
from jax.experimental import pallas as pl
from jax.experimental.pallas import tpu as pltpu
from jax.experimental.pallas import tpu_sc as plsc
from jax.scipy.special import ndtri

D = 64
NW = 32
C = 256
KC = C // 128
G = C // 16
FRAC = np.float32(0.999 * 63 - 62)


def _sc_body(idx_hbm, idxg_hbm, w_hbm, midlut_hbm, flut_hbm, out_hbm,
             idx_v, idxg_v, rows_v, out_v, midlut_v, flut_v, sem, nchunk):
    wid = lax.axis_index("s") * 2 + lax.axis_index("c")
    pltpu.sync_copy(midlut_hbm, midlut_v)
    pltpu.sync_copy(flut_hbm, flut_v)
    iota16 = lax.iota(jnp.int32, 16)
    perms = [iota16 ^ (1 << b) for b in range(4)]
    rpw = nchunk * C

    @pl.loop(0, nchunk)
    def _chunk(g):
        row0 = wid * rpw + g * C
        irow = wid * (rpw // 128) + g * KC
        pltpu.sync_copy(idx_hbm.at[pl.ds(irow, KC)], idx_v)
        pltpu.sync_copy(idxg_hbm.at[pl.ds(irow, KC)], idxg_v)
        cps = [pltpu.async_copy(w_hbm.at[idxg_v.at[j]],
                                rows_v.at[pl.ds(j * 128, 128)], sem)
               for j in range(KC)]
        for cp in cps:
            cp.wait()

        @pl.loop(0, C, unroll=2)
        def _row(r):
            pv = idx_v[r >> 7, pl.ds(r & 0x70, 16)]
            lane = jnp.zeros((16,), jnp.int32) + (r & 15)
            splat = pv.at[lane].get(mode="promise_in_bounds")
            msk = jnp.bitwise_and(splat, 1) == 1
            v = [jnp.where(msk,
                           rows_v[r, pl.ds(64 + 16 * k, 16)],
                           rows_v[r, pl.ds(16 * k, 16)]) for k in range(4)]
            a = [jnp.abs(x) for x in v]
            s1 = jnp.maximum(a[0], a[1])
            t1 = jnp.minimum(a[0], a[1])
            s2 = jnp.maximum(a[2], a[3])
            t2 = jnp.minimum(a[2], a[3])
            m1 = jnp.maximum(s1, s2)
            m2 = jnp.maximum(jnp.minimum(s1, s2), jnp.maximum(t1, t2))
            for p in perms:
                pm1 = m1.at[p].get(mode="promise_in_bounds")
                pm2 = m2.at[p].get(mode="promise_in_bounds")
                m2 = jnp.maximum(jnp.minimum(m1, pm1),
                                 jnp.where(m1 >= pm1, m2, pm2))
                m1 = jnp.maximum(m1, pm1)
            amax = jnp.maximum(m2 + FRAC * (m1 - m2), np.float32(1e-8))
            inv = np.float32(1.0) / amax
            namax = -amax
            ooff = (r & 1) << 6
            for k in range(4):
                xn = jnp.minimum(jnp.maximum(v[k], namax), amax) * inv
                u = jnp.minimum(((xn + np.float32(1.0)) * np.float32(128.0))
                                .astype(jnp.int32), 255)
                mv = plsc.load_gather(midlut_v, [u])
                u2 = u + u + jnp.where(xn > mv, 1, 0)
                q = plsc.load_gather(flut_v, [u2])
                out_v[r >> 1, pl.ds(ooff + 16 * k, 16)] = q * amax

        pltpu.sync_copy(out_v, out_hbm.at[pl.ds(row0 >> 1, C // 2)])


@jax.jit
def _axs_embed(idx2d, idxg2d, weight2, midlut, flut):
    nrows = idx2d.shape[0] * 128
    nchunk = nrows // (NW * C)
    body = functools.partial(_sc_body, nchunk=nchunk)
    f = pl.kernel(
        body,
        out_type=jax.ShapeDtypeStruct((nrows // 2, 128), jnp.float32),
        mesh=plsc.VectorSubcoreMesh(core_axis_name="c", subcore_axis_name="s",
                                    num_cores=2, num_subcores=16),
        scratch_types=[
            pltpu.VMEM((KC, 128), jnp.int32),
            pltpu.VMEM((KC, 128), jnp.int32),
            pltpu.VMEM((C, 128), jnp.float32),
            pltpu.VMEM((C // 2, 128), jnp.float32),
            pltpu.VMEM((256,), jnp.float32),
            pltpu.VMEM((512,), jnp.float32),
            pltpu.SemaphoreType.DMA,
        ],
        compiler_params=pltpu.CompilerParams(needs_layout_passes=False,
                                             use_tc_tiling_on_sc=False),
    )
    return f(idx2d, idxg2d, weight2, midlut, flut)


def kernel(input, weight):
    nrows = input.shape[0] * input.shape[1]
    idx2d = input.reshape(nrows // 128, 128)
    probs = (jnp.arange(32, dtype=jnp.float32) + 0.5) / 32
    lv = ndtri(probs)
    lv = (lv / jnp.max(jnp.abs(lv))).astype(jnp.float32)
    mids = (lv[:-1] + lv[1:]) * np.float32(0.5)
    midpad = jnp.concatenate([mids, jnp.full((1,), 2.0, jnp.float32)])
    edges = jnp.arange(256, dtype=jnp.float32) / np.float32(128.0) - 1
    lut = jnp.sum(mids[None, :] < edges[:, None], axis=1).astype(jnp.int32)
    midlut = midpad[lut]
    flut = lv[jnp.minimum(lut[:, None] + jnp.arange(2)[None, :], 31)].reshape(512)
    weight2 = weight.reshape(weight.shape[0] // 2, 128)
    out = _axs_embed(idx2d, jnp.right_shift(idx2d, 1), weight2, midlut, flut)
    return out.reshape(input.shape[0], input.shape[1], D)


# speedup vs baseline: 1.0002x; 1.0002x over previous
import functools

import jax
import jax.numpy as jnp
import numpy as np
from jax import lax
from jax.experimental import pallas as pl
from jax.experimental.pallas import tpu as pltpu
from jax.experimental.pallas import tpu_sc as plsc
from jax.scipy.special import ndtri

D = 64
NW = 32
C = 256
KC = C // 128
G = C // 16
FRAC = np.float32(0.999 * 63 - 62)


def _sc_body(idx_hbm, idxg_hbm, w_hbm, midlut_hbm, flut_hbm, out_hbm,
             idx_v, idxg_v, rows_v, out_v, midlut_v, flut_v, sem, nchunk):
    wid = lax.axis_index("s") * 2 + lax.axis_index("c")
    pltpu.sync_copy(midlut_hbm, midlut_v)
    pltpu.sync_copy(flut_hbm, flut_v)
    iota16 = lax.iota(jnp.int32, 16)
    perms = [iota16 ^ (1 << b) for b in range(4)]
    rpw = nchunk * C

    @pl.loop(0, nchunk)
    def _chunk(g):
        row0 = pl.multiple_of(wid * rpw + g * C, C)
        pltpu.sync_copy(idx_hbm.at[pl.ds(row0, C)], idx_v)
        pltpu.sync_copy(idxg_hbm.at[pl.ds(row0, C)], idxg_v)
        cps = [pltpu.async_copy(w_hbm.at[idxg_v.at[pl.ds(j * 128, 128)]],
                                rows_v.at[pl.ds(j * 128, 128)], sem)
               for j in range(KC)]
        for cp in cps:
            cp.wait()

        @pl.loop(0, C, unroll=2)
        def _row(r):
            pv = idx_v[pl.ds(r & 0xF0, 16)]
            lane = jnp.zeros((16,), jnp.int32) + (r & 15)
            splat = pv.at[lane].get(mode="promise_in_bounds")
            msk = jnp.bitwise_and(splat, 1) == 1
            v = [jnp.where(msk,
                           rows_v[r, pl.ds(64 + 16 * k, 16)],
                           rows_v[r, pl.ds(16 * k, 16)]) for k in range(4)]
            a = [jnp.abs(x) for x in v]
            s1 = jnp.maximum(a[0], a[1])
            t1 = jnp.minimum(a[0], a[1])
            s2 = jnp.maximum(a[2], a[3])
            t2 = jnp.minimum(a[2], a[3])
            m1 = jnp.maximum(s1, s2)
            m2 = jnp.maximum(jnp.minimum(s1, s2), jnp.maximum(t1, t2))
            for p in perms:
                pm1 = m1.at[p].get(mode="promise_in_bounds")
                pm2 = m2.at[p].get(mode="promise_in_bounds")
                m2 = jnp.maximum(jnp.minimum(m1, pm1),
                                 jnp.where(m1 >= pm1, m2, pm2))
                m1 = jnp.maximum(m1, pm1)
            amax = jnp.maximum(m2 + FRAC * (m1 - m2), np.float32(1e-8))
            inv = np.float32(1.0) / amax
            namax = -amax
            ooff = (r & 1) << 6
            for k in range(4):
                xn = jnp.minimum(jnp.maximum(v[k], namax), amax) * inv
                u = jnp.minimum(((xn + np.float32(1.0)) * np.float32(128.0))
                                .astype(jnp.int32), 255)
                mv = plsc.load_gather(midlut_v, [u])
                u2 = u + u + jnp.where(xn > mv, 1, 0)
                q = plsc.load_gather(flut_v, [u2])
                out_v[r >> 1, pl.ds(ooff + 16 * k, 16)] = q * amax

        pltpu.sync_copy(out_v, out_hbm.at[pl.ds(pl.multiple_of(row0 >> 1, C // 2), C // 2)])


@jax.jit
def _axs_embed(idx2d, idxg2d, weight2, midlut, flut):
    nrows = idx2d.shape[0]
    nchunk = nrows // (NW * C)
    body = functools.partial(_sc_body, nchunk=nchunk)
    f = pl.kernel(
        body,
        out_type=jax.ShapeDtypeStruct((nrows // 2, 128), jnp.float32),
        mesh=plsc.VectorSubcoreMesh(core_axis_name="c", subcore_axis_name="s",
                                    num_cores=2, num_subcores=16),
        scratch_types=[
            pltpu.VMEM((C,), jnp.int32),
            pltpu.VMEM((C,), jnp.int32),
            pltpu.VMEM((C, 128), jnp.float32),
            pltpu.VMEM((C // 2, 128), jnp.float32),
            pltpu.VMEM((256,), jnp.float32),
            pltpu.VMEM((512,), jnp.float32),
            pltpu.SemaphoreType.DMA,
        ],
        compiler_params=pltpu.CompilerParams(needs_layout_passes=False,
                                             use_tc_tiling_on_sc=True),
    )
    return f(idx2d, idxg2d, weight2, midlut, flut)


def kernel(input, weight):
    nrows = input.shape[0] * input.shape[1]
    idx2d = input.reshape(nrows)
    probs = (jnp.arange(32, dtype=jnp.float32) + 0.5) / 32
    lv = ndtri(probs)
    lv = (lv / jnp.max(jnp.abs(lv))).astype(jnp.float32)
    mids = (lv[:-1] + lv[1:]) * np.float32(0.5)
    midpad = jnp.concatenate([mids, jnp.full((1,), 2.0, jnp.float32)])
    edges = jnp.arange(256, dtype=jnp.float32) / np.float32(128.0) - 1
    lut = jnp.sum(mids[None, :] < edges[:, None], axis=1).astype(jnp.int32)
    midlut = midpad[lut]
    flut = lv[jnp.minimum(lut[:, None] + jnp.arange(2)[None, :], 31)].reshape(512)
    weight2 = weight.reshape(weight.shape[0] // 2, 128)
    out = _axs_embed(idx2d, jnp.right_shift(idx2d, 1), weight2, midlut, flut)
    return out.reshape(input.shape[0], input.shape[1], D)


# speedup vs baseline: 1.0822x; 1.0820x over previous
import functools

import jax
import jax.numpy as jnp
import numpy as np
from jax import lax
from jax.experimental import pallas as pl
from jax.experimental.pallas import tpu as pltpu
from jax.experimental.pallas import tpu_sc as plsc
from jax.scipy.special import ndtri

D = 64
NW = 32
CC = 128
FRAC = np.float32(0.999 * 63 - 62)


def _sc_body(idx_hbm, w_hbm, mvs_hbm, flut_hbm, out_hbm,
             idx_v0, idx_v1, rows_v0, rows_v1, out_v0, out_v1,
             mvs_v, flut_v,
             sem_i0, sem_i1, sem_g0, sem_g1, sem_o0, sem_o1, nchunk):
    wid = lax.axis_index("s") * 2 + lax.axis_index("c")
    pltpu.sync_copy(mvs_hbm, mvs_v)
    pltpu.sync_copy(flut_hbm, flut_v)
    iota16 = lax.iota(jnp.int32, 16)
    zero16 = iota16 * 0
    one16 = zero16 + 1
    idx_vs = (idx_v0, idx_v1)
    rows_vs = (rows_v0, rows_v1)
    out_vs = (out_v0, out_v1)
    sem_is = (sem_i0, sem_i1)
    sem_gs = (sem_g0, sem_g1)
    sem_os = (sem_o0, sem_o1)
    rpw = nchunk * CC
    irow0 = wid * (rpw // 128)

    def wait_idx(b):
        pltpu.make_async_copy(idx_hbm.at[pl.ds(0, 1)], idx_vs[b],
                              sem_is[b]).wait()

    def wait_gather(b):
        pltpu.make_async_copy(w_hbm.at[pl.ds(0, CC)], rows_vs[b],
                              sem_gs[b]).wait()

    def wait_out(b):
        pltpu.make_async_copy(out_vs[b], out_hbm.at[pl.ds(0, CC)],
                              sem_os[b]).wait()

    pltpu.async_copy(idx_hbm.at[pl.ds(irow0, 1)], idx_v0, sem_i0)
    pltpu.async_copy(idx_hbm.at[pl.ds(irow0 + 1, 1)], idx_v1, sem_i1)
    wait_idx(0)
    pltpu.async_copy(w_hbm.at[idx_v0.at[0]], rows_v0, sem_g0)

    @pl.loop(0, nchunk, step=2)
    def _pair(g):
        for b in range(2):
            c = g + b
            idx_v, rows_v, out_v = idx_vs[b], rows_vs[b], out_vs[b]

            wait_gather(b)

            @pl.when(c + 2 < nchunk)
            def _pf_idx():
                pltpu.async_copy(idx_hbm.at[pl.ds(irow0 + c + 2, 1)],
                                 idx_v, sem_is[b])

            @pl.when(c + 1 < nchunk)
            def _pf_gather():
                wait_idx(1 - b)
                pltpu.async_copy(w_hbm.at[idx_vs[1 - b].at[0]],
                                 rows_vs[1 - b], sem_gs[1 - b])

            @pl.when(c >= 2)
            def _drain_out():
                wait_out(b)

            @pl.loop(0, CC, unroll=4)
            def _row(r):
                v = [rows_v[r, pl.ds(16 * k, 16)] for k in range(4)]
                a = [jnp.abs(x) for x in v]
                s1 = jnp.maximum(a[0], a[1])
                t1 = jnp.minimum(a[0], a[1])
                s2 = jnp.maximum(a[2], a[3])
                t2 = jnp.minimum(a[2], a[3])
                m1v = jnp.maximum(s1, s2)
                m2v = jnp.maximum(jnp.minimum(s1, s2), jnp.maximum(t1, t2))
                ks, vs = plsc.sort_key_val(m1v, m2v, descending=True)
                m1 = ks.at[zero16].get(mode="promise_in_bounds")
                k1 = ks.at[one16].get(mode="promise_in_bounds")
                v0 = vs.at[zero16].get(mode="promise_in_bounds")
                m2 = jnp.maximum(k1, v0)
                amax = jnp.maximum(m2 + FRAC * (m1 - m2), np.float32(1e-8))
                inv128 = np.float32(128.0) / amax
                namax = -amax
                for k in range(4):
                    x = jnp.minimum(jnp.maximum(v[k], namax), amax)
                    uf = x * inv128 + np.float32(128.0)
                    u = jnp.minimum(uf.astype(jnp.int32), 255)
                    mvs = plsc.load_gather(mvs_v, [u])
                    cbit = jnp.where(uf > mvs, 1, 0)
                    q = plsc.load_gather(flut_v, [u + u + cbit])
                    out_v[r, pl.ds(16 * k, 16)] = q * amax

            row0 = pl.multiple_of(wid * rpw + c * CC, CC)
            pltpu.async_copy(out_v, out_hbm.at[pl.ds(row0, CC)], sem_os[b])

    wait_out(0)
    wait_out(1)


@jax.jit
def _axs_embed(idx2d, weight, mvs, flut):
    nrows = idx2d.shape[0] * 128
    nchunk = nrows // (NW * CC)
    body = functools.partial(_sc_body, nchunk=nchunk)
    f = pl.kernel(
        body,
        out_type=jax.ShapeDtypeStruct((nrows, D), jnp.float32),
        mesh=plsc.VectorSubcoreMesh(core_axis_name="c", subcore_axis_name="s",
                                    num_cores=2, num_subcores=16),
        scratch_types=[
            pltpu.VMEM((1, 128), jnp.int32),
            pltpu.VMEM((1, 128), jnp.int32),
            pltpu.VMEM((CC, D), jnp.float32),
            pltpu.VMEM((CC, D), jnp.float32),
            pltpu.VMEM((CC, D), jnp.float32),
            pltpu.VMEM((CC, D), jnp.float32),
            pltpu.VMEM((256,), jnp.float32),
            pltpu.VMEM((512,), jnp.float32),
            pltpu.SemaphoreType.DMA,
            pltpu.SemaphoreType.DMA,
            pltpu.SemaphoreType.DMA,
            pltpu.SemaphoreType.DMA,
            pltpu.SemaphoreType.DMA,
            pltpu.SemaphoreType.DMA,
        ],
        compiler_params=pltpu.CompilerParams(needs_layout_passes=False,
                                             use_tc_tiling_on_sc=False),
    )
    return f(idx2d, weight, mvs, flut)


def kernel(input, weight):
    nrows = input.shape[0] * input.shape[1]
    idx2d = input.reshape(nrows // 128, 128)
    probs = (jnp.arange(32, dtype=jnp.float32) + 0.5) / 32
    lv = ndtri(probs)
    lv = (lv / jnp.max(jnp.abs(lv))).astype(jnp.float32)
    mids = (lv[:-1] + lv[1:]) * np.float32(0.5)
    midpad = jnp.concatenate([mids, jnp.full((1,), 2.0, jnp.float32)])
    edges = jnp.arange(256, dtype=jnp.float32) / np.float32(128.0) - 1
    lut = jnp.sum(mids[None, :] < edges[:, None], axis=1).astype(jnp.int32)
    mvs = (midpad[lut] + 1) * np.float32(128.0)
    flut = lv[jnp.minimum(lut[:, None] + jnp.arange(2)[None, :], 31)]
    out = _axs_embed(idx2d, weight, mvs, flut.reshape(512))
    return out.reshape(input.shape[0], input.shape[1], D)
